# baseline (speedup 1.0000x reference)
import functools

import jax
import jax.numpy as jnp
from jax import lax
from jax.experimental import pallas as pl
from jax.experimental.pallas import tpu as pltpu, tpu_sc as plsc

N_ATOMS = 100000
EMB = 128

_info = plsc.get_sparse_core_info()
_NC, _NS = _info.num_cores, _info.num_subcores
_NW = _NC * _NS

_CH = 112
_QW = 3128
_Q_LAST = N_ATOMS - (_NW - 1) * _QW
_NFULL = _Q_LAST // _CH
_T_MAIN = _QW - _NFULL * _CH
_T_LAST = _Q_LAST - _NFULL * _CH
_IN_PAD = _QW * _NW

_mesh = plsc.VectorSubcoreMesh(core_axis_name="c", subcore_axis_name="s")


@functools.partial(
    pl.kernel,
    mesh=_mesh,
    out_type=jax.ShapeDtypeStruct((N_ATOMS, EMB), jnp.float32),
    scratch_types=[
        pltpu.VMEM((_QW,), jnp.int32),
        pltpu.VMEM((_CH, EMB), jnp.float32),
        pltpu.VMEM((_CH, EMB), jnp.float32),
        pltpu.SemaphoreType.DMA,
        pltpu.SemaphoreType.DMA,
    ],
)
def _gather_kernel(table_hbm, idx_hbm, out_hbm, idx_v, rows_a, rows_b,
                   sem_a, sem_b):
    wid = lax.axis_index("s") * _NC + lax.axis_index("c")
    base = wid * _QW
    pltpu.sync_copy(idx_hbm.at[pl.ds(base, _QW)], idx_v)
    bufs = (rows_a, rows_b)
    sems = (sem_a, sem_b)
    copies = [None, None]
    copies[0] = pltpu.async_copy(
        table_hbm.at[idx_v.at[pl.ds(0, _CH)]], bufs[0], sems[0])
    for k in range(_NFULL):
        nxt = (k + 1) % 2
        if k + 1 < _NFULL:
            copies[nxt] = pltpu.async_copy(
                table_hbm.at[idx_v.at[pl.ds((k + 1) * _CH, _CH)]],
                bufs[nxt], sems[nxt])
        copies[k % 2].wait()
        pltpu.sync_copy(bufs[k % 2], out_hbm.at[pl.ds(base + k * _CH, _CH)])

    tail_off = _NFULL * _CH

    @pl.when(wid < _NW - 1)
    def _():
        tb = bufs[0].at[pl.ds(0, _T_MAIN)]
        pltpu.async_copy(
            table_hbm.at[idx_v.at[pl.ds(tail_off, _T_MAIN)]],
            tb, sems[0]).wait()
        pltpu.sync_copy(tb, out_hbm.at[pl.ds(base + tail_off, _T_MAIN)])

    @pl.when(wid == _NW - 1)
    def _():
        tb = bufs[1].at[pl.ds(0, _T_LAST)]
        pltpu.async_copy(
            table_hbm.at[idx_v.at[pl.ds(tail_off, _T_LAST)]],
            tb, sems[1]).wait()
        pltpu.sync_copy(tb, out_hbm.at[pl.ds(base + tail_off, _T_LAST)])


def kernel(C, table):
    idx = jnp.pad(C.astype(jnp.int32), (0, _IN_PAD - N_ATOMS))
    return _gather_kernel(table.astype(jnp.float32), idx)


# speedup vs baseline: 12.3715x; 12.3715x over previous
import functools

import jax
import jax.numpy as jnp
from jax import lax
from jax.experimental import pallas as pl
from jax.experimental.pallas import tpu as pltpu, tpu_sc as plsc

N_ATOMS = 100000
EMB = 128
NROWS = 9

_info = plsc.get_sparse_core_info()
_NC, _NS = _info.num_cores, _info.num_subcores
_NW = _NC * _NS

_CH = 112
_QW = 3128
_Q_LAST = N_ATOMS - (_NW - 1) * _QW
_NFULL = _Q_LAST // _CH
_T_MAIN = _QW - _NFULL * _CH
_T_LAST = _Q_LAST - _NFULL * _CH
_IN_PAD = _QW * _NW

_mesh = plsc.VectorSubcoreMesh(core_axis_name="c", subcore_axis_name="s")


@functools.partial(
    pl.kernel,
    mesh=_mesh,
    out_type=jax.ShapeDtypeStruct((N_ATOMS, EMB), jnp.float32),
    scratch_types=[
        pltpu.VMEM_SHARED((NROWS, EMB), jnp.float32),
        pltpu.VMEM((_QW,), jnp.int32),
        pltpu.VMEM((_CH, EMB), jnp.float32),
        pltpu.VMEM((_CH, EMB), jnp.float32),
        pltpu.SemaphoreType.DMA,
        pltpu.SemaphoreType.DMA,
    ],
)
def _gather_kernel(table_hbm, idx_hbm, out_hbm, tab_sh, idx_v, rows_a,
                   rows_b, sem_a, sem_b):
    sid = lax.axis_index("s")
    wid = sid * _NC + lax.axis_index("c")
    base = wid * _QW

    @pl.when(sid == 0)
    def _():
        pltpu.sync_copy(table_hbm, tab_sh)

    pltpu.sync_copy(idx_hbm.at[pl.ds(base, _QW)], idx_v)
    plsc.subcore_barrier()

    bufs = (rows_a, rows_b)
    sems = (sem_a, sem_b)
    copies = [None, None]
    copies[0] = pltpu.async_copy(
        tab_sh.at[idx_v.at[pl.ds(0, _CH)]], bufs[0], sems[0])
    for k in range(_NFULL):
        nxt = (k + 1) % 2
        if k + 1 < _NFULL:
            copies[nxt] = pltpu.async_copy(
                tab_sh.at[idx_v.at[pl.ds((k + 1) * _CH, _CH)]],
                bufs[nxt], sems[nxt])
        copies[k % 2].wait()
        pltpu.sync_copy(bufs[k % 2], out_hbm.at[pl.ds(base + k * _CH, _CH)])

    tail_off = _NFULL * _CH

    @pl.when(wid < _NW - 1)
    def _():
        tb = bufs[0].at[pl.ds(0, _T_MAIN)]
        pltpu.async_copy(
            tab_sh.at[idx_v.at[pl.ds(tail_off, _T_MAIN)]],
            tb, sems[0]).wait()
        pltpu.sync_copy(tb, out_hbm.at[pl.ds(base + tail_off, _T_MAIN)])

    @pl.when(wid == _NW - 1)
    def _():
        tb = bufs[1].at[pl.ds(0, _T_LAST)]
        pltpu.async_copy(
            tab_sh.at[idx_v.at[pl.ds(tail_off, _T_LAST)]],
            tb, sems[1]).wait()
        pltpu.sync_copy(tb, out_hbm.at[pl.ds(base + tail_off, _T_LAST)])


def kernel(C, table):
    idx = jnp.pad(C.astype(jnp.int32), (0, _IN_PAD - N_ATOMS))
    return _gather_kernel(table.astype(jnp.float32), idx)
